# Initial kernel scaffold; baseline (speedup 1.0000x reference)
import jax
import jax.numpy as jnp
from jax.experimental import pallas as pl

NUM_EXPERTS = 64
TOP_K = 8
EPS = 1e-06
BLOCK_M = 512


def _router_body(x_ref, wt_ref, b_ref, o_ref):
    logits = jnp.dot(x_ref[...], wt_ref[...],
                     preferred_element_type=jnp.float32) + b_ref[...]
    m = jnp.max(logits, axis=-1, keepdims=True)
    e = jnp.exp(logits - m)
    z = jnp.sum(e, axis=-1, keepdims=True)
    scores = e / z
    work = logits
    mask = jnp.zeros_like(scores)
    neg = jnp.float32(-jnp.inf)
    for _ in range(TOP_K):
        mx = jnp.max(work, axis=-1, keepdims=True)
        is_max = work == mx
        first = jnp.cumsum(is_max.astype(jnp.int32), axis=-1) == 1
        sel = is_max & first
        mask = mask + sel.astype(scores.dtype)
        work = jnp.where(sel, neg, work)
    masked = scores * mask
    s = jnp.sum(masked, axis=-1, keepdims=True) + EPS
    o_ref[...] = masked / s


@jax.jit
def kernel(x, W, b):
    B, S, D = x.shape
    M = B * S
    x2 = x.reshape(M, D)
    wt = W.T
    b2 = b.reshape(1, NUM_EXPERTS)
    grid = (M // BLOCK_M,)
    out = pl.pallas_call(
        _router_body,
        grid=grid,
        in_specs=[
            pl.BlockSpec((BLOCK_M, D), lambda i: (i, 0)),
            pl.BlockSpec((D, NUM_EXPERTS), lambda i: (0, 0)),
            pl.BlockSpec((1, NUM_EXPERTS), lambda i: (0, 0)),
        ],
        out_specs=pl.BlockSpec((BLOCK_M, NUM_EXPERTS), lambda i: (i, 0)),
        out_shape=jax.ShapeDtypeStruct((M, NUM_EXPERTS), jnp.float32),
    )(x2, wt, b2)
    return out.reshape(B, S, NUM_EXPERTS)


# speedup vs baseline: 1.2345x; 1.2345x over previous
import jax
import jax.numpy as jnp
from jax.experimental import pallas as pl

NUM_EXPERTS = 64
TOP_K = 8
EPS = 1e-06
BLOCK_M = 512


def _router_body(x_ref, wt_ref, b_ref, o_ref):
    logits = jnp.dot(x_ref[...], wt_ref[...],
                     preferred_element_type=jnp.float32) + b_ref[...]
    m = jnp.max(logits, axis=-1, keepdims=True)
    e = jnp.exp(logits - m)
    z = jnp.sum(e, axis=-1, keepdims=True)
    scores = e / z
    lane = jax.lax.broadcasted_iota(jnp.int32, logits.shape, 1)
    work = logits
    mask = jnp.zeros_like(scores)
    neg = jnp.float32(-jnp.inf)
    for _ in range(TOP_K):
        mx = jnp.max(work, axis=-1, keepdims=True)
        is_max = work == mx
        first_idx = jnp.min(jnp.where(is_max, lane, NUM_EXPERTS),
                            axis=-1, keepdims=True)
        sel = lane == first_idx
        mask = mask + sel.astype(scores.dtype)
        work = jnp.where(sel, neg, work)
    masked = scores * mask
    s = jnp.sum(masked, axis=-1, keepdims=True) + EPS
    o_ref[...] = masked / s


@jax.jit
def kernel(x, W, b):
    B, S, D = x.shape
    M = B * S
    x2 = x.reshape(M, D)
    wt = W.T
    b2 = b.reshape(1, NUM_EXPERTS)
    grid = (M // BLOCK_M,)
    out = pl.pallas_call(
        _router_body,
        grid=grid,
        in_specs=[
            pl.BlockSpec((BLOCK_M, D), lambda i: (i, 0)),
            pl.BlockSpec((D, NUM_EXPERTS), lambda i: (0, 0)),
            pl.BlockSpec((1, NUM_EXPERTS), lambda i: (0, 0)),
        ],
        out_specs=pl.BlockSpec((BLOCK_M, NUM_EXPERTS), lambda i: (i, 0)),
        out_shape=jax.ShapeDtypeStruct((M, NUM_EXPERTS), jnp.float32),
    )(x2, wt, b2)
    return out.reshape(B, S, NUM_EXPERTS)


# speedup vs baseline: 1.4139x; 1.1454x over previous
import jax
import jax.numpy as jnp
from jax.experimental import pallas as pl

NUM_EXPERTS = 64
TOP_K = 8
EPS = 1e-06
BLOCK_M = 1024


def _router_body(x_ref, wt_ref, b_ref, o_ref):
    logits = jnp.dot(x_ref[...], wt_ref[...],
                     preferred_element_type=jnp.float32) + b_ref[...]
    m = jnp.max(logits, axis=-1, keepdims=True)
    e = jnp.exp(logits - m)
    z = jnp.sum(e, axis=-1, keepdims=True)
    scores = e / z
    lane = jax.lax.broadcasted_iota(jnp.int32, logits.shape, 1)
    work = logits
    mask = jnp.zeros_like(scores)
    neg = jnp.float32(-jnp.inf)
    for _ in range(TOP_K):
        mx = jnp.max(work, axis=-1, keepdims=True)
        is_max = work == mx
        first_idx = jnp.min(jnp.where(is_max, lane, NUM_EXPERTS),
                            axis=-1, keepdims=True)
        sel = lane == first_idx
        mask = mask + sel.astype(scores.dtype)
        work = jnp.where(sel, neg, work)
    masked = scores * mask
    s = jnp.sum(masked, axis=-1, keepdims=True) + EPS
    o_ref[...] = masked / s


@jax.jit
def kernel(x, W, b):
    B, S, D = x.shape
    M = B * S
    x2 = x.reshape(M, D)
    wt = W.T
    b2 = b.reshape(1, NUM_EXPERTS)
    grid = (M // BLOCK_M,)
    out = pl.pallas_call(
        _router_body,
        grid=grid,
        in_specs=[
            pl.BlockSpec((BLOCK_M, D), lambda i: (i, 0)),
            pl.BlockSpec((D, NUM_EXPERTS), lambda i: (0, 0)),
            pl.BlockSpec((1, NUM_EXPERTS), lambda i: (0, 0)),
        ],
        out_specs=pl.BlockSpec((BLOCK_M, NUM_EXPERTS), lambda i: (i, 0)),
        out_shape=jax.ShapeDtypeStruct((M, NUM_EXPERTS), jnp.float32),
    )(x2, wt, b2)
    return out.reshape(B, S, NUM_EXPERTS)


# speedup vs baseline: 1.4178x; 1.0027x over previous
import jax
import jax.numpy as jnp
from jax.experimental import pallas as pl
from jax.experimental.pallas import tpu as pltpu

NUM_EXPERTS = 64
TOP_K = 8
EPS = 1e-06
BLOCK_M = 1024


def _router_body(x_ref, wt_ref, b_ref, o_ref):
    logits = jnp.dot(x_ref[...], wt_ref[...],
                     preferred_element_type=jnp.float32) + b_ref[...]
    m = jnp.max(logits, axis=-1, keepdims=True)
    e = jnp.exp(logits - m)
    z = jnp.sum(e, axis=-1, keepdims=True)
    scores = e / z
    lane = jax.lax.broadcasted_iota(jnp.int32, logits.shape, 1)
    work = logits
    mask = jnp.zeros_like(scores)
    neg = jnp.float32(-jnp.inf)
    for _ in range(TOP_K):
        mx = jnp.max(work, axis=-1, keepdims=True)
        is_max = work == mx
        first_idx = jnp.min(jnp.where(is_max, lane, NUM_EXPERTS),
                            axis=-1, keepdims=True)
        sel = lane == first_idx
        mask = mask + sel.astype(scores.dtype)
        work = jnp.where(sel, neg, work)
    masked = scores * mask
    s = jnp.sum(masked, axis=-1, keepdims=True) + EPS
    o_ref[...] = masked / s


@jax.jit
def kernel(x, W, b):
    B, S, D = x.shape
    M = B * S
    x2 = x.reshape(M, D)
    wt = W.T
    b2 = b.reshape(1, NUM_EXPERTS)
    grid = (M // BLOCK_M,)
    out = pl.pallas_call(
        _router_body,
        grid=grid,
        in_specs=[
            pl.BlockSpec((BLOCK_M, D), lambda i: (i, 0)),
            pl.BlockSpec((D, NUM_EXPERTS), lambda i: (0, 0)),
            pl.BlockSpec((1, NUM_EXPERTS), lambda i: (0, 0)),
        ],
        out_specs=pl.BlockSpec((BLOCK_M, NUM_EXPERTS), lambda i: (i, 0)),
        out_shape=jax.ShapeDtypeStruct((M, NUM_EXPERTS), jnp.float32),
        compiler_params=pltpu.CompilerParams(
            dimension_semantics=("parallel",)),
    )(x2, wt, b2)
    return out.reshape(B, S, NUM_EXPERTS)


# speedup vs baseline: 1.6934x; 1.1943x over previous
import jax
import jax.numpy as jnp
from jax.experimental import pallas as pl
from jax.experimental.pallas import tpu as pltpu

NUM_EXPERTS = 64
TOP_K = 8
EPS = 1e-06
BLOCK_M = 1024


def _router_body(x_ref, wt_ref, b_ref, o_ref):
    logits = jnp.dot(x_ref[...], wt_ref[...],
                     preferred_element_type=jnp.float32)
    lt = logits.T + b_ref[...]
    m = jnp.max(lt, axis=0, keepdims=True)
    e = jnp.exp(lt - m)
    z = jnp.sum(e, axis=0, keepdims=True)
    work = lt
    neg = jnp.float32(-jnp.inf)
    t = None
    for _ in range(TOP_K):
        t = jnp.max(work, axis=0, keepdims=True)
        work = jnp.where(work == t, neg, work)
    mask = (lt >= t).astype(jnp.float32)
    me = e * mask
    s = jnp.sum(me, axis=0, keepdims=True) + EPS * z
    o_ref[...] = me / s


@jax.jit
def kernel(x, W, b):
    B, S, D = x.shape
    M = B * S
    x2 = x.reshape(M, D)
    wt = W.T
    b2 = b.reshape(NUM_EXPERTS, 1)
    grid = (M // BLOCK_M,)
    out = pl.pallas_call(
        _router_body,
        grid=grid,
        in_specs=[
            pl.BlockSpec((BLOCK_M, D), lambda i: (i, 0)),
            pl.BlockSpec((D, NUM_EXPERTS), lambda i: (0, 0)),
            pl.BlockSpec((NUM_EXPERTS, 1), lambda i: (0, 0)),
        ],
        out_specs=pl.BlockSpec((NUM_EXPERTS, BLOCK_M), lambda i: (0, i)),
        out_shape=jax.ShapeDtypeStruct((NUM_EXPERTS, M), jnp.float32),
        compiler_params=pltpu.CompilerParams(
            dimension_semantics=("arbitrary",)),
    )(x2, wt, b2)
    return out.T.reshape(B, S, NUM_EXPERTS)
